# baseline (speedup 1.0000x reference)
import functools

import jax
import jax.numpy as jnp
from jax import lax
from jax.experimental import pallas as pl
from jax.experimental.pallas import tpu as pltpu
from jax.experimental.pallas import tpu_sc as plsc

T = 2048
DIM = 211
E = 100
HID = 256
DP = 256
EP = 128
NPAIR = 2 * T
RB = 128
BT = 256
NT = NPAIR // BT
NW = NT + E
NSC = 32
TPW = T // NSC
EPS = 1e-5


def _gate_shared_body(x_ref, wg_ref, bg_ref, gs_ref, bs_ref, ws1_ref, bs1_ref,
                      ws3_ref, bs3_ref, ws2_ref, bs2_ref,
                      xhat_ref, z_ref, wbc_ref, itop_ref):
    xb = x_ref[...]
    rows = xb.shape[0]
    lane = lax.broadcasted_iota(jnp.int32, (rows, DP), 1)
    dmask = lane < DIM
    m = jnp.sum(xb, axis=-1, keepdims=True) / DIM
    d = jnp.where(dmask, xb - m, 0.0)
    v = jnp.sum(d * d, axis=-1, keepdims=True) / DIM
    xh = d * lax.rsqrt(v + EPS)
    xhat_ref[...] = xh

    sc = jnp.dot(xb, wg_ref[...], preferred_element_type=jnp.float32)
    s = jax.nn.sigmoid(sc + bg_ref[0, :])
    el = lax.broadcasted_iota(jnp.int32, (rows, EP), 1)
    s = jnp.where(el < E, s, -1.0)
    m1 = jnp.max(s, axis=-1, keepdims=True)
    i1 = jnp.min(jnp.where(s == m1, el, EP), axis=-1, keepdims=True)
    s2 = jnp.where(el == i1, -1.0, s)
    m2 = jnp.max(s2, axis=-1, keepdims=True)
    i2 = jnp.min(jnp.where(s2 == m2, el, EP), axis=-1, keepdims=True)
    tot = m1 + m2
    itop_ref[...] = jnp.where(el == 0, i1, jnp.where(el == 1, i2, 0))
    l32 = lax.broadcasted_iota(jnp.int32, (rows, 32), 1)
    wbc_ref[...] = jnp.where(l32 < 16, m1 / tot, m2 / tot)

    xs = xh * gs_ref[0, :] + bs_ref[0, :]
    h1 = jnp.dot(xs, ws1_ref[...], preferred_element_type=jnp.float32) + bs1_ref[0, :]
    h3 = jnp.dot(xs, ws3_ref[...], preferred_element_type=jnp.float32) + bs3_ref[0, :]
    h = h1 * jax.nn.sigmoid(h1) * h3
    z_ref[...] = jnp.dot(h, ws2_ref[...], preferred_element_type=jnp.float32) + bs2_ref[0, :]


def _gate_shared(xp, wgp, bgp, gsp, bsp, ws1p, bs1r, ws3p, bs3r, ws2p, bs2p):
    r = 256
    grid = (T // r,)
    row_spec = lambda w: pl.BlockSpec((r, w), lambda i: (i, 0))
    full = lambda shape: pl.BlockSpec(shape, lambda i: tuple(0 for _ in shape))
    return pl.pallas_call(
        _gate_shared_body,
        grid=grid,
        in_specs=[row_spec(DP), full((DP, EP)), full((1, EP)),
                  full((1, DP)), full((1, DP)),
                  full((DP, HID)), full((1, HID)),
                  full((DP, HID)), full((1, HID)),
                  full((HID, DP)), full((1, DP))],
        out_specs=[row_spec(DP), row_spec(DP), row_spec(32), row_spec(EP)],
        out_shape=[jax.ShapeDtypeStruct((T, DP), jnp.float32),
                   jax.ShapeDtypeStruct((T, DP), jnp.float32),
                   jax.ShapeDtypeStruct((T, 32), jnp.float32),
                   jax.ShapeDtypeStruct((T, EP), jnp.int32)],
    )(xp, wgp, bgp, gsp, bsp, ws1p, bs1r, ws3p, bs3r, ws2p, bs2p)


NRB = NPAIR // RB


def _route_body(idx_ref, pos_ref, aux_ref):
    lane2 = lax.broadcasted_iota(jnp.int32, (RB, EP), 1)

    def onehot(b):
        ib = idx_ref[b * RB:(b + 1) * RB, :]
        return (ib == lane2).astype(jnp.float32)

    ss = [jnp.sum(onehot(b), axis=0, keepdims=True) for b in range(NRB)]
    s_blk = jnp.concatenate(ss, axis=0)
    rr = lax.broadcasted_iota(jnp.int32, (NRB, NRB), 0)
    cc = lax.broadcasted_iota(jnp.int32, (NRB, NRB), 1)
    l_nt = (cc < rr).astype(jnp.float32)
    pfx = jnp.dot(l_nt, s_blk, preferred_element_type=jnp.float32)

    counts = jnp.sum(s_blk, axis=0, keepdims=True)
    cnt8 = jnp.broadcast_to(counts, (8, EP))
    ru = lax.broadcasted_iota(jnp.int32, (EP, EP), 0)
    cu = lax.broadcasted_iota(jnp.int32, (EP, EP), 1)
    upper = (ru < cu).astype(jnp.float32)
    off8 = jnp.dot(cnt8, upper, preferred_element_type=jnp.float32)
    off_row = off8[0:1, :]

    rl = lax.broadcasted_iota(jnp.int32, (RB, RB), 0)
    cl = lax.broadcasted_iota(jnp.int32, (RB, RB), 1)
    l_bt = (cl < rl).astype(jnp.float32)
    for b in range(NRB):
        ohb = onehot(b)
        cum = jnp.dot(l_bt, ohb, preferred_element_type=jnp.float32)
        val = ohb * (cum + pfx[b:b + 1, :] + off_row)
        posb = jnp.sum(val, axis=1, keepdims=True)
        pos_ref[b * RB:(b + 1) * RB, :] = posb.astype(jnp.int32)

    aux = jnp.concatenate(
        [counts, off_row, jnp.zeros((6, EP), jnp.float32)], axis=0)
    aux_ref[...] = aux.astype(jnp.int32)


def _route(idxf):
    return pl.pallas_call(
        _route_body,
        out_shape=[jax.ShapeDtypeStruct((NPAIR, 1), jnp.int32),
                   jax.ShapeDtypeStruct((8, EP), jnp.int32)],
    )(idxf)


def _dispatch_body(xhat_hbm, pos3_hbm, out_hbm, idx0_v, idx1_v, rows_v,
                   sem0, sem1):
    wid = lax.axis_index("s") * 2 + lax.axis_index("c")
    base = wid * TPW
    pltpu.sync_copy(xhat_hbm.at[pl.ds(base, TPW)], rows_v)
    pltpu.sync_copy(pos3_hbm.at[wid, 0], idx0_v)
    pltpu.sync_copy(pos3_hbm.at[wid, 1], idx1_v)
    c0 = pltpu.async_copy(rows_v, out_hbm.at[idx0_v], sem0)
    c1 = pltpu.async_copy(rows_v, out_hbm.at[idx1_v], sem1)
    c0.wait()
    c1.wait()


def _dispatch(xhat, pos3):
    mesh = plsc.VectorSubcoreMesh(core_axis_name="c", subcore_axis_name="s")
    f = pl.kernel(
        _dispatch_body,
        out_type=jax.ShapeDtypeStruct((NPAIR, DP), jnp.float32),
        mesh=mesh,
        scratch_types=[pltpu.VMEM((TPW,), jnp.int32),
                       pltpu.VMEM((TPW,), jnp.int32),
                       pltpu.VMEM((TPW, DP), jnp.float32),
                       pltpu.SemaphoreType.DMA,
                       pltpu.SemaphoreType.DMA],
    )
    return f(xhat, pos3)


def _group_body(m_ref, xs_ref, w1a_ref, w1b_ref, w3a_ref, w3b_ref,
                w2a_ref, w2b_ref, pk_ref, out_ref):
    i = pl.program_id(0)
    tile = m_ref[0, i]
    rs = m_ref[2, i]
    re = m_ref[3, i]
    prev = m_ref[0, jnp.maximum(i - 1, 0)]
    first = jnp.logical_or(i == 0, tile != prev)

    @pl.when(first)
    def _():
        out_ref[...] = jnp.zeros_like(out_ref)

    @pl.when(re > rs)
    def _():
        x = xs_ref[...][:, :DIM]
        pk = pk_ref[0]
        xe = (x * pk[0, :DIM] + pk[1, :DIM]).astype(jnp.bfloat16)
        mm = functools.partial(jnp.dot, preferred_element_type=jnp.float32)
        bf = lambda r: r[0].astype(jnp.bfloat16)
        h1 = jnp.concatenate(
            [mm(xe, bf(w1a_ref)), mm(xe, bf(w1b_ref))], axis=1) + pk[2, :]
        h3 = jnp.concatenate(
            [mm(xe, bf(w3a_ref)), mm(xe, bf(w3b_ref))], axis=1) + pk[3, :]
        h = (h1 * jax.nn.sigmoid(h1) * h3).astype(jnp.bfloat16)
        yo = (mm(h[:, :128], bf(w2a_ref)) + mm(h[:, 128:], bf(w2b_ref))
              + pk[4, :DIM])
        rid = lax.broadcasted_iota(jnp.int32, (BT, 1), 0)
        mask = jnp.logical_and(rid >= rs, rid < re)
        contrib = jnp.pad(jnp.where(mask, yo, 0.0), ((0, 0), (0, DP - DIM)))
        out_ref[...] += contrib


def _grouped(meta, xs, w1p, w3p, w2p, pk):
    grid_spec = pltpu.PrefetchScalarGridSpec(
        num_scalar_prefetch=1,
        grid=(NW,),
        in_specs=[
            pl.BlockSpec((BT, DP), lambda i, m: (m[0, i], 0)),
            pl.BlockSpec((1, DIM, 128), lambda i, m: (m[1, i], 0, 0)),
            pl.BlockSpec((1, DIM, 128), lambda i, m: (m[1, i], 0, 1)),
            pl.BlockSpec((1, DIM, 128), lambda i, m: (m[1, i], 0, 0)),
            pl.BlockSpec((1, DIM, 128), lambda i, m: (m[1, i], 0, 1)),
            pl.BlockSpec((1, 128, DIM), lambda i, m: (m[1, i], 0, 0)),
            pl.BlockSpec((1, 128, DIM), lambda i, m: (m[1, i], 1, 0)),
            pl.BlockSpec((1, 5, DP), lambda i, m: (m[1, i], 0, 0)),
        ],
        out_specs=pl.BlockSpec((BT, DP), lambda i, m: (m[0, i], 0)),
    )
    return pl.pallas_call(
        _group_body,
        grid_spec=grid_spec,
        out_shape=jax.ShapeDtypeStruct((NPAIR, DP), jnp.float32),
    )(meta, xs, w1p, w1p, w3p, w3p, w2p, w2p, pk)


def _combine_body(ys_hbm, pos3_hbm, wbc_hbm, z_hbm, out_hbm,
                  idx0_v, idx1_v, g0_v, g1_v, z_v, w_v, y_v, sem0, sem1):
    wid = lax.axis_index("s") * 2 + lax.axis_index("c")
    base = wid * TPW
    pltpu.sync_copy(pos3_hbm.at[wid, 0], idx0_v)
    pltpu.sync_copy(pos3_hbm.at[wid, 1], idx1_v)
    c0 = pltpu.async_copy(ys_hbm.at[idx0_v], g0_v, sem0)
    c1 = pltpu.async_copy(ys_hbm.at[idx1_v], g1_v, sem1)
    pltpu.sync_copy(z_hbm.at[pl.ds(base, TPW)], z_v)
    pltpu.sync_copy(wbc_hbm.at[pl.ds(base, TPW)], w_v)
    c0.wait()
    c1.wait()

    def row(r, carry):
        w0 = w_v[r, pl.ds(0, 16)]
        w1 = w_v[r, pl.ds(16, 16)]
        for c in range(DP // 16):
            sl = pl.ds(c * 16, 16)
            y_v[r, sl] = w0 * g0_v[r, sl] + w1 * g1_v[r, sl] + z_v[r, sl]
        return carry

    lax.fori_loop(0, TPW, row, 0)
    pltpu.sync_copy(y_v, out_hbm.at[pl.ds(base, TPW)])


def _combine(ys, pos3, wbc, z):
    mesh = plsc.VectorSubcoreMesh(core_axis_name="c", subcore_axis_name="s")
    f = pl.kernel(
        _combine_body,
        out_type=jax.ShapeDtypeStruct((T, DP), jnp.float32),
        mesh=mesh,
        scratch_types=[pltpu.VMEM((TPW,), jnp.int32),
                       pltpu.VMEM((TPW,), jnp.int32),
                       pltpu.VMEM((TPW, DP), jnp.float32),
                       pltpu.VMEM((TPW, DP), jnp.float32),
                       pltpu.VMEM((TPW, DP), jnp.float32),
                       pltpu.VMEM((TPW, 32), jnp.float32),
                       pltpu.VMEM((TPW, DP), jnp.float32),
                       pltpu.SemaphoreType.DMA,
                       pltpu.SemaphoreType.DMA],
    )
    return f(ys, pos3, wbc, z)


def _work_items(aux):
    cnt = aux[0, :E]
    off = aux[1, :]
    first_tile = off[:E] // BT
    last_tile = (off[:E] + cnt - 1) // BT
    nitems = jnp.where(cnt > 0, last_tile - first_tile + 1, 0)
    istart = jnp.concatenate(
        [jnp.zeros((1,), jnp.int32), jnp.cumsum(nitems).astype(jnp.int32)])
    total = istart[E]
    w = jnp.arange(NW, dtype=jnp.int32)
    e_w = jnp.clip(jnp.searchsorted(istart, w, side="right") - 1, 0, E - 1)
    e_w = e_w.astype(jnp.int32)
    t_w = first_tile[e_w] + (w - istart[e_w])
    seg_lo = jnp.maximum(off[e_w], t_w * BT)
    seg_hi = jnp.minimum(off[e_w] + cnt[e_w], (t_w + 1) * BT)
    validw = w < total
    tile_id = jnp.where(validw, t_w, NT - 1)
    rs = jnp.where(validw, seg_lo - t_w * BT, 0)
    re = jnp.where(validw, seg_hi - t_w * BT, 0)
    return jnp.stack([tile_id, e_w, rs, re]).astype(jnp.int32)


def kernel(x, Wg, bg, gamma, beta, W1, b1, W3, b3, W2, b2,
           gs, bs, Ws1, bs1, Ws3, bs3, Ws2, bs2):
    padd = DP - DIM
    pade = EP - E
    xp = jnp.pad(x, ((0, 0), (0, padd)))
    wgp = jnp.pad(Wg, ((0, padd), (0, pade)))
    bgp = jnp.pad(bg, (0, pade)).reshape(1, EP)
    gsp = jnp.pad(gs, (0, padd)).reshape(1, DP)
    bsp = jnp.pad(bs, (0, padd)).reshape(1, DP)
    ws1p = jnp.pad(Ws1, ((0, padd), (0, 0)))
    ws3p = jnp.pad(Ws3, ((0, padd), (0, 0)))
    ws2p = jnp.pad(Ws2, ((0, 0), (0, padd)))
    bs2p = jnp.pad(bs2, (0, padd)).reshape(1, DP)

    xhat, z, wbc, itop = _gate_shared(
        xp, wgp, bgp, gsp, bsp, ws1p, bs1.reshape(1, HID),
        ws3p, bs3.reshape(1, HID), ws2p, bs2p)

    idxf = itop[:, :2].reshape(NPAIR, 1)
    pos, aux = _route(idxf)
    meta = _work_items(aux)
    pos3 = pos.reshape(T, 2).reshape(NSC, TPW, 2).transpose(0, 2, 1)
    pos3 = jnp.asarray(pos3, jnp.int32)

    xs_sorted = _dispatch(xhat, pos3)

    padv = ((0, 0), (0, padd))
    pk = jnp.stack([jnp.pad(gamma, padv), jnp.pad(beta, padv),
                    b1, b3, jnp.pad(b2, padv)], axis=1)
    ys = _grouped(meta, xs_sorted, W1, W3, W2, pk)
    y = _combine(ys, pos3, wbc, z)
    return y[:, :DIM]


# speedup vs baseline: 1.2364x; 1.2364x over previous
import functools

import jax
import jax.numpy as jnp
from jax import lax
from jax.experimental import pallas as pl
from jax.experimental.pallas import tpu as pltpu
from jax.experimental.pallas import tpu_sc as plsc

T = 2048
DIM = 211
E = 100
HID = 256
DP = 256
EP = 128
NPAIR = 2 * T
RB = 128
NRB = NPAIR // RB
BT = 256
NT = NPAIR // BT
NW = NT + E
NSC = 32
TPW = T // NSC
EPS = 1e-5


def _front_body(x_ref, wg_ref, bg_ref, gs_ref, bs_ref, ws1_ref, bs1_ref,
                ws3_ref, bs3_ref, ws2_ref, bs2_ref,
                xhat_ref, z_ref, wbc_ref, pos_ref,
                mt_ref, me_ref, mrs_ref, mre_ref):
    xb = x_ref[...]
    m = jnp.sum(xb, axis=-1, keepdims=True) / DIM
    d = xb - m
    v = jnp.sum(d * d, axis=-1, keepdims=True) / DIM
    xh = d * lax.rsqrt(v + EPS)
    xhat_ref[...] = jnp.pad(xh, ((0, 0), (0, DP - DIM)))

    sc = jnp.dot(xb, wg_ref[...], preferred_element_type=jnp.float32)
    s = jax.nn.sigmoid(sc + bg_ref[0, :])
    el = lax.broadcasted_iota(jnp.int32, (T, EP), 1)
    s = jnp.where(el < E, s, -1.0)
    m1 = jnp.max(s, axis=-1, keepdims=True)
    i1 = jnp.min(jnp.where(s == m1, el, EP), axis=-1, keepdims=True)
    s2 = jnp.where(el == i1, -1.0, s)
    m2 = jnp.max(s2, axis=-1, keepdims=True)
    i2 = jnp.min(jnp.where(s2 == m2, el, EP), axis=-1, keepdims=True)
    tot = m1 + m2
    l32 = lax.broadcasted_iota(jnp.int32, (T, 32), 1)
    wbc_ref[...] = jnp.where(l32 < 16, m1 / tot, m2 / tot)

    xs = xh * gs_ref[0, :] + bs_ref[0, :]
    h1 = jnp.dot(xs, ws1_ref[...], preferred_element_type=jnp.float32) + bs1_ref[0, :]
    h3 = jnp.dot(xs, ws3_ref[...], preferred_element_type=jnp.float32) + bs3_ref[0, :]
    h = h1 * jax.nn.sigmoid(h1) * h3
    z = jnp.dot(h, ws2_ref[...], preferred_element_type=jnp.float32) + bs2_ref[0, :]
    z_ref[...] = jnp.pad(z, ((0, 0), (0, DP - DIM)))

    lane2 = lax.broadcasted_iota(jnp.int32, (RB, EP), 1)
    nhalf = T // RB

    def onehot(b):
        src = i1 if b < nhalf else i2
        ib = src[(b % nhalf) * RB:((b % nhalf) + 1) * RB, :]
        return (ib == lane2).astype(jnp.float32)

    ss = [jnp.sum(onehot(b), axis=0, keepdims=True) for b in range(NRB)]
    s_blk = jnp.concatenate(ss, axis=0)
    rr = lax.broadcasted_iota(jnp.int32, (NRB, NRB), 0)
    cc = lax.broadcasted_iota(jnp.int32, (NRB, NRB), 1)
    l_nrb = (cc < rr).astype(jnp.float32)
    pfx = jnp.dot(l_nrb, s_blk, preferred_element_type=jnp.float32)

    counts = jnp.sum(s_blk, axis=0, keepdims=True)
    cnt8 = jnp.broadcast_to(counts, (8, EP))
    ru = lax.broadcasted_iota(jnp.int32, (EP, EP), 0)
    cu = lax.broadcasted_iota(jnp.int32, (EP, EP), 1)
    upper = (ru < cu).astype(jnp.float32)
    off8 = jnp.dot(cnt8, upper, preferred_element_type=jnp.float32)
    off_row = off8[0:1, :]

    rl = lax.broadcasted_iota(jnp.int32, (RB, RB), 0)
    cl = lax.broadcasted_iota(jnp.int32, (RB, RB), 1)
    l_rb = (cl < rl).astype(jnp.float32)
    for b in range(NRB):
        ohb = onehot(b)
        cum = jnp.dot(l_rb, ohb, preferred_element_type=jnp.float32)
        val = ohb * (cum + pfx[b:b + 1, :] + off_row)
        posb = jnp.sum(val, axis=1, keepdims=True)
        pos_ref[b * RB:(b + 1) * RB, :] = posb.astype(jnp.int32)

    ft = jnp.floor(off_row / BT)
    lt = jnp.floor((off_row + counts - 1.0) / BT)
    ni = jnp.where(counts > 0, lt - ft + 1.0, 0.0)
    ni8 = jnp.broadcast_to(ni, (8, EP))
    ist8 = jnp.dot(ni8, upper, preferred_element_type=jnp.float32)
    ist_row = ist8[0:1, :]
    lane_row = lax.broadcasted_iota(jnp.int32, (1, EP), 1)
    ist_cmp = jnp.where(lane_row <= E, ist_row, 1e9)
    ist_b = jnp.broadcast_to(ist_cmp, (EP, EP))
    w_col = lax.broadcasted_iota(jnp.int32, (EP, 1), 0).astype(jnp.float32)
    e_w = jnp.sum((ist_b <= w_col).astype(jnp.float32), axis=1,
                  keepdims=True) - 1.0
    e_w = jnp.minimum(e_w, E - 1.0)
    lane_b = lax.broadcasted_iota(jnp.int32, (EP, EP), 1).astype(jnp.float32)
    ohw = (lane_b == e_w).astype(jnp.float32)

    def gath(row):
        return jnp.sum(ohw * row, axis=1, keepdims=True)

    ist_g = gath(jnp.broadcast_to(ist_row, (EP, EP)))
    ft_g = gath(jnp.broadcast_to(ft, (EP, EP)))
    off_g = gath(jnp.broadcast_to(off_row, (EP, EP)))
    cnt_g = gath(jnp.broadcast_to(counts, (EP, EP)))
    t_w = ft_g + (w_col - ist_g)
    lo = jnp.maximum(off_g, t_w * BT)
    hi = jnp.minimum(off_g + cnt_g, (t_w + 1.0) * BT)
    tot_w = jnp.sum(jnp.where(lane_b == E, jnp.broadcast_to(ist_row, (EP, EP)),
                              0.0), axis=1, keepdims=True)
    valid = w_col < tot_w
    mt_ref[...] = jnp.where(valid, t_w, NT - 1.0).astype(jnp.int32)
    me_ref[...] = e_w.astype(jnp.int32)
    mrs_ref[...] = jnp.where(valid, lo - t_w * BT, 0.0).astype(jnp.int32)
    mre_ref[...] = jnp.where(valid, hi - t_w * BT, 0.0).astype(jnp.int32)


def _front(x, wgp, bgp, gs, bs, ws1, bs1r, ws3, bs3r, ws2, bs2r):
    sds = jax.ShapeDtypeStruct
    return pl.pallas_call(
        _front_body,
        out_shape=[sds((T, DP), jnp.float32), sds((T, DP), jnp.float32),
                   sds((T, 32), jnp.float32), sds((NPAIR, 1), jnp.int32),
                   sds((EP, 1), jnp.int32), sds((EP, 1), jnp.int32),
                   sds((EP, 1), jnp.int32), sds((EP, 1), jnp.int32)],
    )(x, wgp, bgp, gs, bs, ws1, bs1r, ws3, bs3r, ws2, bs2r)


def _dispatch_body(xhat_hbm, pos3_hbm, out_hbm, idx0_v, idx1_v, rows_v,
                   sem0, sem1):
    wid = lax.axis_index("s") * 2 + lax.axis_index("c")
    base = wid * TPW
    pltpu.sync_copy(xhat_hbm.at[pl.ds(base, TPW)], rows_v)
    pltpu.sync_copy(pos3_hbm.at[wid, 0], idx0_v)
    pltpu.sync_copy(pos3_hbm.at[wid, 1], idx1_v)
    c0 = pltpu.async_copy(rows_v, out_hbm.at[idx0_v], sem0)
    c1 = pltpu.async_copy(rows_v, out_hbm.at[idx1_v], sem1)
    c0.wait()
    c1.wait()


def _dispatch(xhat, pos3):
    mesh = plsc.VectorSubcoreMesh(core_axis_name="c", subcore_axis_name="s")
    f = pl.kernel(
        _dispatch_body,
        out_type=jax.ShapeDtypeStruct((NPAIR, DP), jnp.float32),
        mesh=mesh,
        scratch_types=[pltpu.VMEM((TPW,), jnp.int32),
                       pltpu.VMEM((TPW,), jnp.int32),
                       pltpu.VMEM((TPW, DP), jnp.float32),
                       pltpu.SemaphoreType.DMA,
                       pltpu.SemaphoreType.DMA],
    )
    return f(xhat, pos3)


def _group_body(mt_ref, me_ref, mrs_ref, mre_ref,
                xs_ref, w1_ref, w3_ref, w2_ref, pk_ref, out_ref):
    i = pl.program_id(0)
    tile = mt_ref[i, 0]
    rs = mrs_ref[i, 0]
    re = mre_ref[i, 0]
    prev = mt_ref[jnp.maximum(i - 1, 0), 0]
    first = jnp.logical_or(i == 0, tile != prev)

    @pl.when(first)
    def _():
        out_ref[...] = jnp.zeros_like(out_ref)

    @pl.when(re > rs)
    def _():
        x = xs_ref[...][:, :DIM]
        pk = pk_ref[0]
        xe = (x * pk[0, :DIM] + pk[1, :DIM]).astype(jnp.bfloat16)
        w1 = w1_ref[0].astype(jnp.bfloat16)
        w3 = w3_ref[0].astype(jnp.bfloat16)
        h1 = jnp.dot(xe, w1, preferred_element_type=jnp.float32) + pk[2, :]
        h3 = jnp.dot(xe, w3, preferred_element_type=jnp.float32) + pk[3, :]
        h = (h1 * jax.nn.sigmoid(h1) * h3).astype(jnp.bfloat16)
        w2 = w2_ref[0].astype(jnp.bfloat16)
        yo = jnp.dot(h, w2, preferred_element_type=jnp.float32) + pk[4, :DIM]
        rid = lax.broadcasted_iota(jnp.int32, (BT, 1), 0)
        mask = jnp.logical_and(rid >= rs, rid < re)
        contrib = jnp.pad(jnp.where(mask, yo, 0.0), ((0, 0), (0, DP - DIM)))
        out_ref[...] += contrib


def _grouped(mt, me, mrs, mre, xs, w1p, w3p, w2p, pk):
    grid_spec = pltpu.PrefetchScalarGridSpec(
        num_scalar_prefetch=4,
        grid=(NW,),
        in_specs=[
            pl.BlockSpec((BT, DP), lambda i, mt, me, s, e: (mt[i, 0], 0)),
            pl.BlockSpec((1, DIM, HID), lambda i, mt, me, s, e: (me[i, 0], 0, 0)),
            pl.BlockSpec((1, DIM, HID), lambda i, mt, me, s, e: (me[i, 0], 0, 0)),
            pl.BlockSpec((1, HID, DIM), lambda i, mt, me, s, e: (me[i, 0], 0, 0)),
            pl.BlockSpec((1, 5, DP), lambda i, mt, me, s, e: (me[i, 0], 0, 0)),
        ],
        out_specs=pl.BlockSpec((BT, DP), lambda i, mt, me, s, e: (mt[i, 0], 0)),
    )
    return pl.pallas_call(
        _group_body,
        grid_spec=grid_spec,
        out_shape=jax.ShapeDtypeStruct((NPAIR, DP), jnp.float32),
    )(mt, me, mrs, mre, xs, w1p, w3p, w2p, pk)


def _combine_body(ys_hbm, pos3_hbm, wbc_hbm, z_hbm, out_hbm,
                  idx0_v, idx1_v, g0_v, g1_v, z_v, w_v, y_v, sem0, sem1):
    wid = lax.axis_index("s") * 2 + lax.axis_index("c")
    base = wid * TPW
    pltpu.sync_copy(pos3_hbm.at[wid, 0], idx0_v)
    pltpu.sync_copy(pos3_hbm.at[wid, 1], idx1_v)
    c0 = pltpu.async_copy(ys_hbm.at[idx0_v], g0_v, sem0)
    c1 = pltpu.async_copy(ys_hbm.at[idx1_v], g1_v, sem1)
    pltpu.sync_copy(z_hbm.at[pl.ds(base, TPW)], z_v)
    pltpu.sync_copy(wbc_hbm.at[pl.ds(base, TPW)], w_v)
    c0.wait()
    c1.wait()

    def row(r, carry):
        w0 = w_v[r, pl.ds(0, 16)]
        w1 = w_v[r, pl.ds(16, 16)]
        for c in range(DP // 16):
            sl = pl.ds(c * 16, 16)
            y_v[r, sl] = w0 * g0_v[r, sl] + w1 * g1_v[r, sl] + z_v[r, sl]
        return carry

    lax.fori_loop(0, TPW, row, 0)
    pltpu.sync_copy(y_v, out_hbm.at[pl.ds(base, TPW)])


def _combine(ys, pos3, wbc, z):
    mesh = plsc.VectorSubcoreMesh(core_axis_name="c", subcore_axis_name="s")
    f = pl.kernel(
        _combine_body,
        out_type=jax.ShapeDtypeStruct((T, DP), jnp.float32),
        mesh=mesh,
        scratch_types=[pltpu.VMEM((TPW,), jnp.int32),
                       pltpu.VMEM((TPW,), jnp.int32),
                       pltpu.VMEM((TPW, DP), jnp.float32),
                       pltpu.VMEM((TPW, DP), jnp.float32),
                       pltpu.VMEM((TPW, DP), jnp.float32),
                       pltpu.VMEM((TPW, 32), jnp.float32),
                       pltpu.VMEM((TPW, DP), jnp.float32),
                       pltpu.SemaphoreType.DMA,
                       pltpu.SemaphoreType.DMA],
    )
    return f(ys, pos3, wbc, z)


def kernel(x, Wg, bg, gamma, beta, W1, b1, W3, b3, W2, b2,
           gs, bs, Ws1, bs1, Ws3, bs3, Ws2, bs2):
    pade = EP - E
    wgp = jnp.pad(Wg, ((0, 0), (0, pade)))
    bgp = jnp.pad(bg, (0, pade)).reshape(1, EP)

    xhat, z, wbc, pos, mt, me, mrs, mre = _front(
        x, wgp, bgp, gs.reshape(1, DIM), bs.reshape(1, DIM),
        Ws1, bs1.reshape(1, HID), Ws3, bs3.reshape(1, HID),
        Ws2, bs2.reshape(1, DIM))

    pos0 = pos[:T].reshape(NSC, TPW)
    pos1 = pos[T:].reshape(NSC, TPW)
    pos3 = jnp.stack([pos0, pos1], axis=1)

    xs_sorted = _dispatch(xhat, pos3)

    padv = ((0, 0), (0, DP - DIM))
    pk = jnp.stack([jnp.pad(gamma, padv), jnp.pad(beta, padv),
                    b1, b3, jnp.pad(b2, padv)], axis=1)
    ys = _grouped(mt, me, mrs, mre, xs_sorted, W1, W3, W2, pk)
    y = _combine(ys, pos3, wbc, z)
    return y[:, :DIM]
